# baseline (speedup 1.0000x reference)
import functools

import jax
import jax.numpy as jnp
from jax import lax
from jax.experimental import pallas as pl
from jax.experimental.pallas import tpu as pltpu
from jax.experimental.pallas import tpu_sc as plsc

N_FIELDS = 26
VOCAB = 100000
DIM = 32
BATCH = 16384

_info = plsc.get_sparse_core_info()
_NC = _info.num_cores
_NS = _info.num_subcores
_NW = _NC * _NS

_WB = 8192
_L = 16
_K = 8

_mesh = plsc.VectorSubcoreMesh(core_axis_name="c", subcore_axis_name="s")


@functools.partial(
    pl.kernel,
    mesh=_mesh,
    out_type=jax.ShapeDtypeStruct((N_FIELDS, DIM, BATCH), jnp.float32),
    scratch_types=[
        pltpu.VMEM((BATCH,), jnp.int32),
        pltpu.VMEM((VOCAB,), jnp.float32),
        pltpu.VMEM((_WB,), jnp.float32),
    ],
    compiler_params=pltpu.CompilerParams(
        use_tc_tiling_on_sc=True, needs_layout_passes=False
    ),
)
def _sc_gather(valt_hbm, tabt_hbm, out_hbm, idx_v, row_v, g_v):
    d = lax.axis_index("s") * _NC + lax.axis_index("c")

    def field_body(f, _):
        pltpu.sync_copy(valt_hbm.at[f], idx_v)
        pltpu.sync_copy(tabt_hbm.at[f, d], row_v)

        def piece_body(c, _):
            def vec_body(i, _):
                base = i * (_K * _L)
                ivs = [idx_v[pl.ds(c * _WB + base + k * _L, _L)] for k in range(_K)]
                gs = [plsc.load_gather(row_v, [iv]) for iv in ivs]
                for k in range(_K):
                    g_v[pl.ds(base + k * _L, _L)] = gs[k]
                return ()

            lax.fori_loop(0, _WB // (_K * _L), vec_body, (), unroll=2)
            pltpu.sync_copy(g_v, out_hbm.at[f, d, pl.ds(c * _WB, _WB)])
            return ()

        lax.fori_loop(0, BATCH // _WB, piece_body, ())
        return ()

    lax.fori_loop(0, N_FIELDS, field_body, ())


def kernel(value, tables):
    valt = value.astype(jnp.int32).T
    tabt = jnp.transpose(tables, (0, 2, 1))
    out = _sc_gather(valt, tabt)
    return jnp.transpose(out, (2, 0, 1))


# speedup vs baseline: 1.0321x; 1.0321x over previous
import functools

import jax
import jax.numpy as jnp
from jax import lax
from jax.experimental import pallas as pl
from jax.experimental.pallas import tpu as pltpu
from jax.experimental.pallas import tpu_sc as plsc

N_FIELDS = 26
VOCAB = 100000
DIM = 32
BATCH = 16384

_info = plsc.get_sparse_core_info()
_NC = _info.num_cores
_NS = _info.num_subcores
_NW = _NC * _NS

_WB = 8192
_L = 16
_K = 16

_mesh = plsc.VectorSubcoreMesh(core_axis_name="c", subcore_axis_name="s")


@functools.partial(
    pl.kernel,
    mesh=_mesh,
    out_type=jax.ShapeDtypeStruct((N_FIELDS, DIM, BATCH), jnp.float32),
    scratch_types=[
        pltpu.VMEM((BATCH,), jnp.int32),
        pltpu.VMEM((VOCAB,), jnp.float32),
        pltpu.VMEM((_WB,), jnp.float32),
    ],
    compiler_params=pltpu.CompilerParams(
        use_tc_tiling_on_sc=True, needs_layout_passes=False
    ),
)
def _sc_gather(valt_hbm, tabt_hbm, out_hbm, idx_v, row_v, g_v):
    d = lax.axis_index("s") * _NC + lax.axis_index("c")

    def field_body(j, _):
        f = lax.rem(d + j, N_FIELDS)
        pltpu.sync_copy(valt_hbm.at[f], idx_v)
        pltpu.sync_copy(tabt_hbm.at[f, d], row_v)

        def piece_body(c, _):
            def vec_body(i, _):
                base = i * (_K * _L)
                ivs = [idx_v[pl.ds(c * _WB + base + k * _L, _L)] for k in range(_K)]
                gs = [plsc.load_gather(row_v, [iv]) for iv in ivs]
                for k in range(_K):
                    g_v[pl.ds(base + k * _L, _L)] = gs[k]
                return ()

            lax.fori_loop(0, _WB // (_K * _L), vec_body, (), unroll=2)
            pltpu.sync_copy(g_v, out_hbm.at[f, d, pl.ds(c * _WB, _WB)])
            return ()

        lax.fori_loop(0, BATCH // _WB, piece_body, ())
        return ()

    lax.fori_loop(0, N_FIELDS, field_body, ())


def kernel(value, tables):
    valt = value.astype(jnp.int32).T
    tabt = jnp.transpose(tables, (0, 2, 1))
    out = _sc_gather(valt, tabt)
    return jnp.transpose(out, (2, 0, 1))


# speedup vs baseline: 1.0688x; 1.0356x over previous
import functools

import jax
import jax.numpy as jnp
from jax import lax
from jax.experimental import pallas as pl
from jax.experimental.pallas import tpu as pltpu
from jax.experimental.pallas import tpu_sc as plsc

N_FIELDS = 26
VOCAB = 100000
DIM = 32
BATCH = 16384

_info = plsc.get_sparse_core_info()
_NC = _info.num_cores
_NS = _info.num_subcores
_NW = _NC * _NS

_WB = 8192
_L = 16
_K = 16

_mesh = plsc.VectorSubcoreMesh(core_axis_name="c", subcore_axis_name="s")


@functools.partial(
    pl.kernel,
    mesh=_mesh,
    out_type=jax.ShapeDtypeStruct((N_FIELDS, DIM, BATCH), jnp.float32),
    scratch_types=[
        pltpu.VMEM((BATCH,), jnp.int32),
        pltpu.VMEM((VOCAB,), jnp.float32),
        pltpu.VMEM((_WB,), jnp.float32),
        pltpu.SemaphoreType.DMA,
        pltpu.SemaphoreType.DMA,
    ],
    compiler_params=pltpu.CompilerParams(
        use_tc_tiling_on_sc=True, needs_layout_passes=False
    ),
)
def _sc_gather(vflat_hbm, tabt_hbm, out_hbm, idx_v, row_v, g_v, sem_i, sem_r):
    d = lax.axis_index("s") * _NC + lax.axis_index("c")

    def field_body(j, _):
        f = lax.rem(d + j, N_FIELDS)
        cp_i = pltpu.async_copy(
            vflat_hbm.at[pl.ds(f * BATCH, BATCH)], idx_v, sem_i
        )
        cp_r = pltpu.async_copy(tabt_hbm.at[f, d], row_v, sem_r)
        cp_i.wait()
        cp_r.wait()

        def piece_body(c, _):
            def vec_body(i, _):
                base = i * (_K * _L)
                ivs = [idx_v[pl.ds(c * _WB + base + k * _L, _L)] for k in range(_K)]
                gs = [plsc.load_gather(row_v, [iv]) for iv in ivs]
                for k in range(_K):
                    g_v[pl.ds(base + k * _L, _L)] = gs[k]
                return ()

            lax.fori_loop(0, _WB // (_K * _L), vec_body, (), unroll=2)
            pltpu.sync_copy(g_v, out_hbm.at[f, d, pl.ds(c * _WB, _WB)])
            return ()

        lax.fori_loop(0, BATCH // _WB, piece_body, ())
        return ()

    lax.fori_loop(0, N_FIELDS, field_body, ())


def kernel(value, tables):
    vflat = value.astype(jnp.int32).T.reshape(N_FIELDS * BATCH)
    tabt = jnp.transpose(tables, (0, 2, 1))
    out = _sc_gather(vflat, tabt)
    return jnp.transpose(out, (2, 0, 1))


# speedup vs baseline: 1.1331x; 1.0601x over previous
import functools

import jax
import jax.numpy as jnp
from jax import lax
from jax.experimental import pallas as pl
from jax.experimental.pallas import tpu as pltpu
from jax.experimental.pallas import tpu_sc as plsc

N_FIELDS = 26
VOCAB = 100000
DIM = 32
BATCH = 16384

_info = plsc.get_sparse_core_info()
_NC = _info.num_cores
_NS = _info.num_subcores
_NW = _NC * _NS

_WB = 4096
_L = 16
_K = 16

_mesh = plsc.VectorSubcoreMesh(core_axis_name="c", subcore_axis_name="s")


@functools.partial(
    pl.kernel,
    mesh=_mesh,
    out_type=jax.ShapeDtypeStruct((N_FIELDS, DIM, BATCH), jnp.float32),
    scratch_types=[
        pltpu.VMEM((BATCH,), jnp.int32),
        pltpu.VMEM((VOCAB,), jnp.float32),
        pltpu.VMEM((_WB,), jnp.float32),
        pltpu.VMEM((_WB,), jnp.float32),
        pltpu.SemaphoreType.DMA,
        pltpu.SemaphoreType.DMA,
        pltpu.SemaphoreType.DMA,
    ],
    compiler_params=pltpu.CompilerParams(
        use_tc_tiling_on_sc=True, needs_layout_passes=False
    ),
)
def _sc_gather(vflat_hbm, tabt_hbm, out_hbm, idx_v, row_v, g0_v, g1_v,
               sem_i, sem_r, sem_w):
    d = lax.axis_index("s") * _NC + lax.axis_index("c")

    def field_body(j, _):
        f = lax.rem(d + j, N_FIELDS)
        cp_i = pltpu.async_copy(
            vflat_hbm.at[pl.ds(f * BATCH, BATCH)], idx_v, sem_i
        )
        cp_r = pltpu.async_copy(tabt_hbm.at[f, d], row_v, sem_r)
        cp_i.wait()
        cp_r.wait()

        gbufs = (g0_v, g1_v)
        wb_cps = []
        for c in range(BATCH // _WB):
            g_v = gbufs[c % 2]
            if c >= 2:
                wb_cps[c - 2].wait()

            def vec_body(i, _, c=c, g_v=g_v):
                base = i * (_K * _L)
                ivs = [idx_v[pl.ds(c * _WB + base + k * _L, _L)] for k in range(_K)]
                gs = [plsc.load_gather(row_v, [iv]) for iv in ivs]
                for k in range(_K):
                    g_v[pl.ds(base + k * _L, _L)] = gs[k]
                return ()

            lax.fori_loop(0, _WB // (_K * _L), vec_body, (), unroll=2)
            wb_cps.append(pltpu.async_copy(
                g_v, out_hbm.at[f, d, pl.ds(c * _WB, _WB)], sem_w))
        wb_cps[-2].wait()
        wb_cps[-1].wait()
        return ()

    lax.fori_loop(0, N_FIELDS, field_body, ())


def kernel(value, tables):
    vflat = value.astype(jnp.int32).T.reshape(N_FIELDS * BATCH)
    tabt = jnp.transpose(tables, (0, 2, 1))
    out = _sc_gather(vflat, tabt)
    return jnp.transpose(out, (2, 0, 1))
